# baseline (speedup 1.0000x reference)
import functools

import jax
import jax.numpy as jnp
from jax import lax
from jax.experimental import pallas as pl
from jax.experimental.pallas import tpu as pltpu
from jax.experimental.pallas import tpu_sc as plsc

N = 100000
D_X = 128
D_E = 12
D_OUT = D_E + D_X
S_PAD = 17

NUM_CORES = 2
NUM_SUBCORES = 16
NW = NUM_CORES * NUM_SUBCORES

PER_W = 3200
N_PAD = NW * PER_W

TC_BLOCK = 10000


def _sc_gather(residue_p, table17):
    mesh = plsc.VectorSubcoreMesh(core_axis_name="c", subcore_axis_name="s")

    @functools.partial(
        pl.kernel,
        mesh=mesh,
        out_type=jax.ShapeDtypeStruct((N_PAD, D_E), jnp.float32),
        scratch_types=[
            pltpu.VMEM((PER_W,), jnp.int32),
            pltpu.VMEM((20, S_PAD), jnp.float32),
            pltpu.VMEM((PER_W, D_E), jnp.float32),
        ],
        compiler_params=pltpu.CompilerParams(
            use_tc_tiling_on_sc=False, needs_layout_passes=False
        ),
    )
    def k(res_hbm, tab_hbm, out_hbm, idx_v, tab_v, rows_v):
        wid = lax.axis_index("s") * NUM_CORES + lax.axis_index("c")
        pltpu.sync_copy(tab_hbm, tab_v)
        pltpu.sync_copy(res_hbm.at[pl.ds(wid * PER_W, PER_W)], idx_v)

        lanes = lax.iota(jnp.int32, 16)

        def group(g, carry):
            idx16 = idx_v[pl.ds(g * 16, 16)]
            row_ids = g * 16 + lanes
            for c in range(D_E):
                csplat = jnp.full((16,), c, jnp.int32)
                vals = plsc.load_gather(tab_v, [idx16, csplat])
                plsc.store_scatter(rows_v, [row_ids, csplat], vals)
            return carry

        lax.fori_loop(0, PER_W // 16, group, 0)

        pltpu.sync_copy(rows_v, out_hbm.at[pl.ds(wid * PER_W, PER_W), :])

    return k(residue_p, table17)


def _tc_concat(emb, x):
    grid = (N // TC_BLOCK,)

    def body(emb_ref, x_ref, o_ref):
        o_ref[...] = jnp.concatenate([emb_ref[...], x_ref[...]], axis=1)

    return pl.pallas_call(
        body,
        grid=grid,
        in_specs=[
            pl.BlockSpec((TC_BLOCK, D_E), lambda i: (i, 0)),
            pl.BlockSpec((TC_BLOCK, D_X), lambda i: (i, 0)),
        ],
        out_specs=pl.BlockSpec((TC_BLOCK, D_OUT), lambda i: (i, 0)),
        out_shape=jax.ShapeDtypeStruct((N, D_OUT), jnp.float32),
    )(emb, x)


def kernel(residue, x, embed_weight):
    table17 = jnp.zeros((embed_weight.shape[0], S_PAD), jnp.float32)
    table17 = table17.at[:, :D_E].set(embed_weight)
    residue_p = jnp.zeros((N_PAD,), jnp.int32).at[:N].set(residue)

    emb = _sc_gather(residue_p, table17)
    return _tc_concat(emb, x)
